# baseline (speedup 1.0000x reference)
import jax
import jax.numpy as jnp
from jax.experimental import pallas as pl
from jax.experimental.pallas import tpu as pltpu

_E = 8


def _moe_body(x_ref, Wg_ref, bg_ref, W_ref, b_ref, out_ref, Wb_ref):
    t = pl.program_id(0)

    @pl.when(t == 0)
    def _():
        for e in range(_E):
            Wb_ref[e] = W_ref[e].astype(jnp.bfloat16)

    xt = x_ref[...]
    logits = (
        jnp.dot(xt, Wg_ref[...], preferred_element_type=jnp.float32)
        + bg_ref[...]
    )
    m = jnp.max(logits, axis=-1, keepdims=True)
    ex = jnp.exp(logits - m)
    wts = ex / jnp.sum(ex, axis=-1, keepdims=True)
    iota = jax.lax.broadcasted_iota(jnp.int32, wts.shape, 1)
    v1 = jnp.max(wts, axis=-1, keepdims=True)
    i1 = jnp.min(jnp.where(wts == v1, iota, _E), axis=-1, keepdims=True)
    rest = jnp.where(iota == i1, -1.0, wts)
    v2 = jnp.max(rest, axis=-1, keepdims=True)
    i2 = jnp.min(jnp.where(rest == v2, iota, _E), axis=-1, keepdims=True)
    c = jnp.where(iota == i1, v1, 0.0) + jnp.where(iota == i2, v2, 0.0)

    xb = xt.astype(jnp.bfloat16)
    acc = jnp.dot(c, b_ref[...], preferred_element_type=jnp.float32)
    for e in range(_E):
        ce = jnp.sum(jnp.where(iota == e, c, 0.0), axis=-1, keepdims=True)
        acc = acc + ce * jnp.dot(
            xb, Wb_ref[e], preferred_element_type=jnp.float32
        )
    out_ref[...] = acc


def kernel(x, Wg, bg, W, b):
    T, D = x.shape
    E = W.shape[0]
    TILE = 256
    return pl.pallas_call(
        _moe_body,
        grid=(T // TILE,),
        in_specs=[
            pl.BlockSpec((TILE, D), lambda t: (t, 0)),
            pl.BlockSpec((D, E), lambda t: (0, 0)),
            pl.BlockSpec((1, E), lambda t: (0, 0)),
            pl.BlockSpec((E, D, D), lambda t: (0, 0, 0)),
            pl.BlockSpec((E, D), lambda t: (0, 0)),
        ],
        out_specs=pl.BlockSpec((TILE, D), lambda t: (t, 0)),
        out_shape=jax.ShapeDtypeStruct((T, D), jnp.float32),
        scratch_shapes=[pltpu.VMEM((E, D, D), jnp.bfloat16)],
        compiler_params=pltpu.CompilerParams(
            dimension_semantics=("arbitrary",),
        ),
    )(x, Wg, bg.reshape(1, E), W, b)


# speedup vs baseline: 1.0826x; 1.0826x over previous
import jax
import jax.numpy as jnp
from jax.experimental import pallas as pl
from jax.experimental.pallas import tpu as pltpu


def _moe_body(x_ref, Wg_ref, bg_ref, W_ref, b_ref, out_ref, c_ref, outT_ref):
    e = pl.program_id(0)
    E = c_ref.shape[0]

    @pl.when(e == 0)
    def _():
        logits = jax.lax.dot_general(
            Wg_ref[...], x_ref[...],
            dimension_numbers=(((0,), (1,)), ((), ())),
            preferred_element_type=jnp.float32,
        ) + bg_ref[...]
        m = jnp.max(logits, axis=0, keepdims=True)
        ex = jnp.exp(logits - m)
        wts = ex / jnp.sum(ex, axis=0, keepdims=True)
        iota = jax.lax.broadcasted_iota(jnp.int32, wts.shape, 0)
        v1 = jnp.max(wts, axis=0, keepdims=True)
        i1 = jnp.min(jnp.where(wts == v1, iota, E), axis=0, keepdims=True)
        rest = jnp.where(iota == i1, -1.0, wts)
        v2 = jnp.max(rest, axis=0, keepdims=True)
        i2 = jnp.min(jnp.where(rest == v2, iota, E), axis=0, keepdims=True)
        c = jnp.where(iota == i1, v1, 0.0) + jnp.where(iota == i2, v2, 0.0)
        c_ref[...] = c
        outT_ref[...] = jax.lax.dot_general(
            b_ref[...], c,
            dimension_numbers=(((0,), (0,)), ((), ())),
            preferred_element_type=jnp.float32,
        )

    c_all = c_ref[...]
    row = jax.lax.broadcasted_iota(jnp.int32, c_all.shape, 0)
    ce = jnp.sum(jnp.where(row == e, c_all, 0.0), axis=0, keepdims=True)
    Mt = jax.lax.dot_general(
        W_ref[0].astype(jnp.bfloat16),
        x_ref[...].astype(jnp.bfloat16),
        dimension_numbers=(((0,), (1,)), ((), ())),
        preferred_element_type=jnp.float32,
    )
    outT_ref[...] += ce * Mt

    @pl.when(e == pl.num_programs(0) - 1)
    def _():
        out_ref[...] = outT_ref[...].T


def kernel(x, Wg, bg, W, b):
    T, D = x.shape
    E = W.shape[0]
    return pl.pallas_call(
        _moe_body,
        grid=(E,),
        in_specs=[
            pl.BlockSpec((T, D), lambda e: (0, 0)),
            pl.BlockSpec((D, E), lambda e: (0, 0)),
            pl.BlockSpec((E, 1), lambda e: (0, 0)),
            pl.BlockSpec((1, D, D), lambda e: (e, 0, 0)),
            pl.BlockSpec((E, D), lambda e: (0, 0)),
        ],
        out_specs=pl.BlockSpec((T, D), lambda e: (0, 0)),
        out_shape=jax.ShapeDtypeStruct((T, D), jnp.float32),
        scratch_shapes=[
            pltpu.VMEM((E, T), jnp.float32),
            pltpu.VMEM((D, T), jnp.float32),
        ],
        compiler_params=pltpu.CompilerParams(
            dimension_semantics=("arbitrary",),
        ),
    )(x, Wg, bg.reshape(E, 1), W, b)
